# Initial kernel scaffold; baseline (speedup 1.0000x reference)
import jax
import jax.numpy as jnp
from jax.experimental import pallas as pl


def kernel(x, W_enc, vectors, W_dec, b_dec):
    raise NotImplementedError("write your pallas kernel here")


# speedup vs baseline: 1.1723x; 1.1723x over previous
import functools

import jax
import jax.numpy as jnp
from jax.experimental import pallas as pl
from jax.experimental.pallas import tpu as pltpu

_BETA = 0.25
_ROW_TILE = 512
_K_TILE = 2048


def _vq_kernel(x_ref, we_ref, v_ref, wd_ref, b_ref, recon_ref, commit_ref):
    x = x_ref[...]
    z = jnp.dot(x, we_ref[...], preferred_element_type=jnp.float32)
    n_kt = v_ref.shape[0] // _K_TILE
    rows = x.shape[0]

    def scan_body(j, carry):
        run_max, run_idx = carry
        vt = v_ref[pl.ds(j * _K_TILE, _K_TILE), :]
        v2 = jnp.sum(vt * vt, axis=1)
        prod = jax.lax.dot_general(
            z, vt, (((1,), (1,)), ((), ())),
            preferred_element_type=jnp.float32)
        score = 2.0 * prod - v2[None, :]
        local_max = jnp.max(score, axis=1)
        local_arg = jnp.argmax(score, axis=1).astype(jnp.int32)
        better = local_max > run_max
        run_idx = jnp.where(better, local_arg + j * _K_TILE, run_idx)
        run_max = jnp.maximum(run_max, local_max)
        return run_max, run_idx

    init = (jnp.full((rows,), -jnp.inf, jnp.float32),
            jnp.zeros((rows,), jnp.int32))
    _, best_idx = jax.lax.fori_loop(0, n_kt, scan_body, init)

    def gather_body(j, acc):
        vt = v_ref[pl.ds(j * _K_TILE, _K_TILE), :]
        iota = jax.lax.broadcasted_iota(jnp.int32, (rows, _K_TILE), 1)
        onehot = (best_idx[:, None] == iota + j * _K_TILE).astype(jnp.float32)
        return acc + jnp.dot(onehot, vt, preferred_element_type=jnp.float32)

    q = jax.lax.fori_loop(0, n_kt, gather_body,
                          jnp.zeros((rows, z.shape[1]), jnp.float32))

    z_q = z + (q - z)
    mean = jnp.dot(z_q, wd_ref[...], preferred_element_type=jnp.float32)
    mean = mean + b_ref[...]
    r = x - mean
    recon_part = jnp.sum(r * r)
    dq = q - z
    commit_part = jnp.sum(dq * dq)

    @pl.when(pl.program_id(0) == 0)
    def _init():
        recon_ref[0, 0] = recon_part
        commit_ref[0, 0] = commit_part

    @pl.when(pl.program_id(0) != 0)
    def _acc():
        recon_ref[0, 0] += recon_part
        commit_ref[0, 0] += commit_part


@jax.jit
def kernel(x, W_enc, vectors, W_dec, b_dec):
    B, T, D_IN = x.shape
    K, D_CODE = vectors.shape
    rows = B * T
    x2 = x.reshape(rows, D_IN)
    b2 = b_dec.reshape(1, D_IN)
    n_row_tiles = rows // _ROW_TILE

    recon_sum, commit_sum = pl.pallas_call(
        _vq_kernel,
        grid=(n_row_tiles,),
        in_specs=[
            pl.BlockSpec((_ROW_TILE, D_IN), lambda i: (i, 0)),
            pl.BlockSpec((D_IN, D_CODE), lambda i: (0, 0)),
            pl.BlockSpec((K, D_CODE), lambda i: (0, 0)),
            pl.BlockSpec((D_CODE, D_IN), lambda i: (0, 0)),
            pl.BlockSpec((1, D_IN), lambda i: (0, 0)),
        ],
        out_specs=[
            pl.BlockSpec(memory_space=pltpu.SMEM),
            pl.BlockSpec(memory_space=pltpu.SMEM),
        ],
        out_shape=[
            jax.ShapeDtypeStruct((1, 1), jnp.float32),
            jax.ShapeDtypeStruct((1, 1), jnp.float32),
        ],
    )(x2, W_enc, vectors, W_dec, b2)

    recon = 0.5 * recon_sum[0, 0] / rows + 0.5 * D_IN * jnp.log(2.0 * jnp.pi)
    commit = commit_sum[0, 0] / (rows * D_CODE)
    return recon + _BETA * commit


# speedup vs baseline: 1.7134x; 1.4616x over previous
import functools

import jax
import jax.numpy as jnp
from jax.experimental import pallas as pl
from jax.experimental.pallas import tpu as pltpu

_BETA = 0.25
_ROW_TILE = 512
_K_TILE = 2048


def _vq_kernel(x_ref, we_ref, v_ref, wd_ref, b_ref, recon_ref, commit_ref,
               v2_ref, vs_ref):
    K = v_ref.shape[0]
    n_kt = K // _K_TILE
    rows = x_ref.shape[0]

    @pl.when(pl.program_id(0) == 0)
    def _prep():
        v = v_ref[...]
        vs_ref[...] = v + v
        v2_ref[...] = jnp.sum(v * v, axis=1)[None, :]

    x = x_ref[...]
    z = jnp.dot(x, we_ref[...], preferred_element_type=jnp.float32)

    def scan_body(j, run):
        vst = vs_ref[pl.ds(j * _K_TILE, _K_TILE), :]
        prod = jax.lax.dot_general(
            z, vst, (((1,), (1,)), ((), ())),
            preferred_element_type=jnp.float32)
        score = prod - v2_ref[:, pl.ds(j * _K_TILE, _K_TILE)]
        enc = (K - 1 - j * _K_TILE) - jax.lax.broadcasted_iota(
            jnp.int32, (1, _K_TILE), 1)
        si = jax.lax.bitcast_convert_type(score, jnp.int32)
        packed = jnp.bitwise_or(jnp.bitwise_and(si, -8192), enc)
        pf = jax.lax.bitcast_convert_type(packed, jnp.float32)
        return jnp.maximum(run, jnp.max(pf, axis=1))

    run = jax.lax.fori_loop(0, n_kt, scan_body,
                            jnp.full((rows,), -jnp.inf, jnp.float32))
    pi = jax.lax.bitcast_convert_type(run, jnp.int32)
    best_idx = (K - 1) - jnp.bitwise_and(pi, 8191)

    def gather_body(j, acc):
        vt = v_ref[pl.ds(j * _K_TILE, _K_TILE), :]
        iota = jax.lax.broadcasted_iota(jnp.int32, (rows, _K_TILE), 1)
        onehot = (best_idx[:, None] == iota + j * _K_TILE).astype(jnp.float32)
        return acc + jnp.dot(onehot, vt, preferred_element_type=jnp.float32)

    q = jax.lax.fori_loop(0, n_kt, gather_body,
                          jnp.zeros((rows, z.shape[1]), jnp.float32))

    z_q = z + (q - z)
    mean = jnp.dot(z_q, wd_ref[...], preferred_element_type=jnp.float32)
    mean = mean + b_ref[...]
    r = x - mean
    recon_part = jnp.sum(r * r)
    dq = q - z
    commit_part = jnp.sum(dq * dq)

    @pl.when(pl.program_id(0) == 0)
    def _init():
        recon_ref[0, 0] = recon_part
        commit_ref[0, 0] = commit_part

    @pl.when(pl.program_id(0) != 0)
    def _acc():
        recon_ref[0, 0] += recon_part
        commit_ref[0, 0] += commit_part


@jax.jit
def kernel(x, W_enc, vectors, W_dec, b_dec):
    B, T, D_IN = x.shape
    K, D_CODE = vectors.shape
    rows = B * T
    x2 = x.reshape(rows, D_IN)
    b2 = b_dec.reshape(1, D_IN)
    n_row_tiles = rows // _ROW_TILE

    recon_sum, commit_sum = pl.pallas_call(
        _vq_kernel,
        grid=(n_row_tiles,),
        in_specs=[
            pl.BlockSpec((_ROW_TILE, D_IN), lambda i: (i, 0)),
            pl.BlockSpec((D_IN, D_CODE), lambda i: (0, 0)),
            pl.BlockSpec((K, D_CODE), lambda i: (0, 0)),
            pl.BlockSpec((D_CODE, D_IN), lambda i: (0, 0)),
            pl.BlockSpec((1, D_IN), lambda i: (0, 0)),
        ],
        out_specs=[
            pl.BlockSpec(memory_space=pltpu.SMEM),
            pl.BlockSpec(memory_space=pltpu.SMEM),
        ],
        out_shape=[
            jax.ShapeDtypeStruct((1, 1), jnp.float32),
            jax.ShapeDtypeStruct((1, 1), jnp.float32),
        ],
        scratch_shapes=[
            pltpu.VMEM((1, K), jnp.float32),
            pltpu.VMEM((K, D_CODE), jnp.float32),
        ],
    )(x2, W_enc, vectors, W_dec, b2)

    recon = 0.5 * recon_sum[0, 0] / rows + 0.5 * D_IN * jnp.log(2.0 * jnp.pi)
    commit = commit_sum[0, 0] / (rows * D_CODE)
    return recon + _BETA * commit
